# Initial kernel scaffold; baseline (speedup 1.0000x reference)
import jax
import jax.numpy as jnp
from jax.experimental import pallas as pl


def kernel(x, edge_index, ss_w1, ss_b1, ss_ln_g, ss_ln_b, ss_w2, ss_b2, cheb2_w, cheb2_b, cheb3_w, cheb3_b, cheb4_w, cheb4_b, att_w, att_b):
    raise NotImplementedError("write your pallas kernel here")


# speedup vs baseline: 6.0806x; 6.0806x over previous
import functools
import math

import jax
import jax.numpy as jnp
from jax import lax
from jax.experimental import pallas as pl
from jax.experimental.pallas import tpu as pltpu
from jax.experimental.pallas import tpu_sc as plsc

N = 10000
E = 320000
D = 128
NPAD = 10240
CHUNK = 128
NCHUNKS = E // CHUNK
NC = 2
NS = 16
NW = NC * NS
TRIPS = -(-NCHUNKS // NW)
ROWS_PER_SUB = NPAD // NS


def _gather2_body(a_hbm, b_hbm, ridx_hbm, cidx_hbm, ga_hbm, gb_hbm,
                  riv, civ, bufa, bufb, sema, semb):
    w = lax.axis_index("s") * NC + lax.axis_index("c")

    def body(i, _):
        cid = w + NW * i

        @pl.when(cid < NCHUNKS)
        def _():
            base = cid * CHUNK
            pltpu.sync_copy(ridx_hbm.at[pl.ds(base, CHUNK)], riv)
            pltpu.sync_copy(cidx_hbm.at[pl.ds(base, CHUNK)], civ)
            cpa = pltpu.async_copy(a_hbm.at[riv], bufa, sema)
            cpb = pltpu.async_copy(b_hbm.at[civ], bufb, semb)
            cpa.wait()
            cpb.wait()
            pltpu.sync_copy(bufa, ga_hbm.at[pl.ds(base, CHUNK)])
            pltpu.sync_copy(bufb, gb_hbm.at[pl.ds(base, CHUNK)])
        return 0

    lax.fori_loop(0, TRIPS, body, 0)


def _deg_body(ridx_hbm, ew_hbm, zeros_hbm, dp_hbm, riv, valv, acc):
    c = lax.axis_index("c")
    s = lax.axis_index("s")
    w = s * NC + c

    pltpu.sync_copy(zeros_hbm.at[pl.ds(s * ROWS_PER_SUB, ROWS_PER_SUB)],
                    acc.at[pl.ds(s * ROWS_PER_SUB, ROWS_PER_SUB)])
    plsc.subcore_barrier()

    def body(i, _):
        cid = w + NW * i

        @pl.when(cid < NCHUNKS)
        def _():
            base = cid * CHUNK
            pltpu.sync_copy(ridx_hbm.at[pl.ds(base, CHUNK)], riv)
            pltpu.sync_copy(ew_hbm.at[pl.ds(base, CHUNK)], valv)
            pltpu.sync_copy(valv, acc.at[riv], add=True)
        return 0

    lax.fori_loop(0, TRIPS, body, 0)
    plsc.subcore_barrier()
    pltpu.sync_copy(acc.at[pl.ds(s * ROWS_PER_SUB, ROWS_PER_SUB)],
                    dp_hbm.at[c, pl.ds(s * ROWS_PER_SUB, ROWS_PER_SUB)])


def _spmm_body(u_hbm, ridx_hbm, cidx_hbm, ew_hbm, zeros_hbm, out_hbm,
               riv, civ, wv, buf, acc, sem):
    c = lax.axis_index("c")
    s = lax.axis_index("s")
    w = s * NC + c

    pltpu.sync_copy(zeros_hbm.at[pl.ds(s * ROWS_PER_SUB, ROWS_PER_SUB)],
                    acc.at[pl.ds(s * ROWS_PER_SUB, ROWS_PER_SUB)])
    plsc.subcore_barrier()

    def body(i, _):
        cid = w + NW * i

        @pl.when(cid < NCHUNKS)
        def _():
            base = cid * CHUNK
            pltpu.sync_copy(ridx_hbm.at[pl.ds(base, CHUNK)], riv)
            pltpu.sync_copy(ew_hbm.at[pl.ds(base, CHUNK)], wv)
            pltpu.async_copy(u_hbm.at[riv], buf, sem).wait()

            def scale(g, _):
                wvec = wv[pl.ds(g * 16, 16)]
                for l in range(16):
                    we = wvec[l]
                    e = g * 16 + l
                    for j in range(D // 16):
                        sl = pl.ds(j * 16, 16)
                        buf[e, sl] = buf[e, sl] * we
                return 0

            lax.fori_loop(0, CHUNK // 16, scale, 0)
            pltpu.sync_copy(cidx_hbm.at[pl.ds(base, CHUNK)], civ)
            pltpu.sync_copy(buf, acc.at[civ], add=True)
        return 0

    lax.fori_loop(0, TRIPS, body, 0)
    plsc.subcore_barrier()
    pltpu.sync_copy(acc.at[pl.ds(s * ROWS_PER_SUB, ROWS_PER_SUB)],
                    out_hbm.at[c, pl.ds(s * ROWS_PER_SUB, ROWS_PER_SUB)])


@functools.lru_cache(maxsize=1)
def _sc_kernels():
    mesh = plsc.VectorSubcoreMesh(core_axis_name="c", subcore_axis_name="s",
                                  num_cores=NC, num_subcores=NS)
    gather2 = pl.kernel(
        _gather2_body,
        out_type=(
            jax.ShapeDtypeStruct((E, D), jnp.float32),
            jax.ShapeDtypeStruct((E, D), jnp.float32),
        ),
        mesh=mesh,
        scratch_types=[
            pltpu.VMEM((CHUNK,), jnp.int32),
            pltpu.VMEM((CHUNK,), jnp.int32),
            pltpu.VMEM((CHUNK, D), jnp.float32),
            pltpu.VMEM((CHUNK, D), jnp.float32),
            pltpu.SemaphoreType.DMA,
            pltpu.SemaphoreType.DMA,
        ],
    )
    deg = pl.kernel(
        _deg_body,
        out_type=jax.ShapeDtypeStruct((NC, NPAD), jnp.float32),
        mesh=mesh,
        scratch_types=[
            pltpu.VMEM((CHUNK,), jnp.int32),
            pltpu.VMEM((CHUNK,), jnp.float32),
            pltpu.VMEM_SHARED((NPAD,), jnp.float32),
        ],
    )
    spmm = pl.kernel(
        _spmm_body,
        out_type=jax.ShapeDtypeStruct((NC, NPAD, D), jnp.float32),
        mesh=mesh,
        scratch_types=[
            pltpu.VMEM((CHUNK,), jnp.int32),
            pltpu.VMEM((CHUNK,), jnp.int32),
            pltpu.VMEM((CHUNK,), jnp.float32),
            pltpu.VMEM((CHUNK, D), jnp.float32),
            pltpu.VMEM_SHARED((NPAD, D), jnp.float32),
            pltpu.SemaphoreType.DMA,
        ],
    )
    return gather2, deg, spmm


_NBLK = 1000
_EBLK = 2000


def _rsqrt_precise(v):
    r = lax.rsqrt(v)
    return r * (1.5 - 0.5 * v * r * r)


def _tc_ab_body(x_ref, w1a_ref, w1b_ref, b1_ref, a_ref, b_ref):
    x = x_ref[...]
    a_ref[...] = jnp.dot(x, w1a_ref[...], preferred_element_type=jnp.float32,
                         precision=lax.Precision.HIGHEST) + b1_ref[...]
    b_ref[...] = jnp.dot(x, w1b_ref[...], preferred_element_type=jnp.float32,
                         precision=lax.Precision.HIGHEST)


def _tc_ab(x, w1a, w1b, b1):
    grid = (N // _NBLK,)
    return pl.pallas_call(
        _tc_ab_body,
        grid=grid,
        in_specs=[
            pl.BlockSpec((_NBLK, D), lambda i: (i, 0)),
            pl.BlockSpec((D, D), lambda i: (0, 0)),
            pl.BlockSpec((D, D), lambda i: (0, 0)),
            pl.BlockSpec((1, D), lambda i: (0, 0)),
        ],
        out_specs=[
            pl.BlockSpec((_NBLK, D), lambda i: (i, 0)),
            pl.BlockSpec((_NBLK, D), lambda i: (i, 0)),
        ],
        out_shape=[
            jax.ShapeDtypeStruct((N, D), jnp.float32),
            jax.ShapeDtypeStruct((N, D), jnp.float32),
        ],
    )(x, w1a, w1b, b1)


def _tc_edge_body(ga_ref, gb_ref, g_ref, b_ref, w2_ref, b2_ref, s_ref):
    h = ga_ref[...] + gb_ref[...]
    mu = jnp.mean(h, axis=-1, keepdims=True)
    hc = h - mu
    var = jnp.mean(hc * hc, axis=-1, keepdims=True)
    hn = hc * _rsqrt_precise(var + 1e-5) * g_ref[...] + b_ref[...]
    hg = 0.5 * hn * (1.0 + lax.erf(hn * (1.0 / math.sqrt(2.0))))
    s = jnp.sum(hg * w2_ref[...], axis=-1, keepdims=True) + b2_ref[...]
    s_ref[...] = jax.nn.sigmoid(s)


def _tc_edge(ga, gb, ln_g, ln_b, w2row, b2):
    grid = (E // _EBLK,)
    return pl.pallas_call(
        _tc_edge_body,
        grid=grid,
        in_specs=[
            pl.BlockSpec((_EBLK, D), lambda i: (i, 0)),
            pl.BlockSpec((_EBLK, D), lambda i: (i, 0)),
            pl.BlockSpec((1, D), lambda i: (0, 0)),
            pl.BlockSpec((1, D), lambda i: (0, 0)),
            pl.BlockSpec((1, D), lambda i: (0, 0)),
            pl.BlockSpec((1, 1), lambda i: (0, 0)),
        ],
        out_specs=pl.BlockSpec((_EBLK, 1), lambda i: (i, 0)),
        out_shape=jax.ShapeDtypeStruct((E, 1), jnp.float32),
    )(ga, gb, ln_g, ln_b, w2row, b2)


def _tc_deg_body(dp0_ref, dp1_ref, x_ref, dis_ref, u_ref):
    deg = dp0_ref[...] + dp1_ref[...]
    pos = deg > 0
    dis = jnp.where(pos, _rsqrt_precise(jnp.where(pos, deg, 1.0)), 0.0)
    dis_ref[...] = dis
    u_ref[...] = x_ref[...] * dis


def _tc_deg(dp0, dp1, x):
    grid = (N // _NBLK,)
    return pl.pallas_call(
        _tc_deg_body,
        grid=grid,
        in_specs=[
            pl.BlockSpec((_NBLK, 1), lambda i: (i, 0)),
            pl.BlockSpec((_NBLK, 1), lambda i: (i, 0)),
            pl.BlockSpec((_NBLK, D), lambda i: (i, 0)),
        ],
        out_specs=[
            pl.BlockSpec((_NBLK, 1), lambda i: (i, 0)),
            pl.BlockSpec((_NBLK, D), lambda i: (i, 0)),
        ],
        out_shape=[
            jax.ShapeDtypeStruct((N, 1), jnp.float32),
            jax.ShapeDtypeStruct((N, D), jnp.float32),
        ],
    )(dp0, dp1, x)


def _tc_comb_body(alpha, beta, p0_ref, p1_ref, dis_ref, prev_ref, t_ref, u_ref):
    dis = dis_ref[...]
    t = alpha * dis * (p0_ref[...] + p1_ref[...]) + beta * prev_ref[...]
    t_ref[...] = t
    u_ref[...] = dis * t


def _tc_comb(alpha, beta, p0, p1, dis, prev):
    grid = (N // _NBLK,)
    return pl.pallas_call(
        functools.partial(_tc_comb_body, alpha, beta),
        grid=grid,
        in_specs=[
            pl.BlockSpec((_NBLK, D), lambda i: (i, 0)),
            pl.BlockSpec((_NBLK, D), lambda i: (i, 0)),
            pl.BlockSpec((_NBLK, 1), lambda i: (i, 0)),
            pl.BlockSpec((_NBLK, D), lambda i: (i, 0)),
        ],
        out_specs=[
            pl.BlockSpec((_NBLK, D), lambda i: (i, 0)),
            pl.BlockSpec((_NBLK, D), lambda i: (i, 0)),
        ],
        out_shape=[
            jax.ShapeDtypeStruct((N, D), jnp.float32),
            jax.ShapeDtypeStruct((N, D), jnp.float32),
        ],
    )(p0, p1, dis, prev)


def _tc_final_body(x_ref, t1_ref, p2a_ref, p2b_ref, p3a_ref, p3b_ref,
                   dis_ref, w2_ref, b2_ref, w3_ref, b3_ref, w4_ref, b4_ref,
                   awt_ref, ab_ref, out_ref):
    dis = dis_ref[...]
    t0 = x_ref[...]
    t1 = t1_ref[...]
    t2 = -2.0 * dis * (p2a_ref[...] + p2b_ref[...]) - t0
    t3 = -2.0 * dis * (p3a_ref[...] + p3b_ref[...]) - t1

    def mm(a, w):
        return jnp.dot(a, w, preferred_element_type=jnp.float32,
                       precision=lax.Precision.HIGHEST)

    f2 = mm(t0, w2_ref[0]) + mm(t1, w2_ref[1]) + b2_ref[...]
    f3 = mm(t0, w3_ref[0]) + mm(t1, w3_ref[1]) + mm(t2, w3_ref[2]) + b3_ref[...]
    f4 = (mm(t0, w4_ref[0]) + mm(t1, w4_ref[1]) + mm(t2, w4_ref[2])
          + mm(t3, w4_ref[3]) + b4_ref[...])

    awt = awt_ref[...]
    fs = (f2, f3, f4)
    s = []
    for j in range(3):
        acc = ab_ref[0, j]
        tot = None
        for k in range(3):
            term = fs[k] * awt[j:j + 1, k * D:(k + 1) * D]
            tot = term if tot is None else tot + term
        s.append(jnp.sum(tot, axis=-1, keepdims=True) + acc)
    m = jnp.maximum(jnp.maximum(s[0], s[1]), s[2])
    e0 = jnp.exp(s[0] - m)
    e1 = jnp.exp(s[1] - m)
    e2 = jnp.exp(s[2] - m)
    out_ref[...] = (f2 * e0 + f3 * e1 + f4 * e2) / (e0 + e1 + e2)


def _tc_final(x, t1, p2a, p2b, p3a, p3b, dis, w2, b2, w3, b3, w4, b4, awt, ab):
    grid = (N // _NBLK,)
    full = lambda shape: pl.BlockSpec(shape, lambda i: tuple(0 for _ in shape))
    blk = pl.BlockSpec((_NBLK, D), lambda i: (i, 0))
    return pl.pallas_call(
        _tc_final_body,
        grid=grid,
        in_specs=[
            blk, blk, blk, blk, blk, blk,
            pl.BlockSpec((_NBLK, 1), lambda i: (i, 0)),
            full((2, D, D)), full((1, D)),
            full((3, D, D)), full((1, D)),
            full((4, D, D)), full((1, D)),
            full((3, 3 * D)),
            pl.BlockSpec((1, 3), lambda i: (0, 0), memory_space=pltpu.SMEM),
        ],
        out_specs=blk,
        out_shape=jax.ShapeDtypeStruct((N, D), jnp.float32),
    )(x, t1, p2a, p2b, p3a, p3b, dis, w2, b2, w3, b3, w4, b4, awt, ab)


def kernel(x, edge_index, ss_w1, ss_b1, ss_ln_g, ss_ln_b, ss_w2, ss_b2,
           cheb2_w, cheb2_b, cheb3_w, cheb3_b, cheb4_w, cheb4_b, att_w, att_b):
    row = edge_index[0]
    col = edge_index[1]
    sc_gather2, sc_deg, sc_spmm = _sc_kernels()

    a_nodes, b_nodes = _tc_ab(x, ss_w1[:D], ss_w1[D:], ss_b1.reshape(1, D))
    ga, gb = sc_gather2(a_nodes, b_nodes, row, col)
    edge_scores = _tc_edge(ga, gb, ss_ln_g.reshape(1, D), ss_ln_b.reshape(1, D),
                           ss_w2.reshape(1, D), ss_b2.reshape(1, 1))
    ew = edge_scores.reshape(E)

    zeros1 = jnp.zeros((NPAD,), jnp.float32)
    zeros2 = jnp.zeros((NPAD, D), jnp.float32)

    dp = sc_deg(row, ew, zeros1)
    dis, u1 = _tc_deg(dp[0, :N].reshape(N, 1), dp[1, :N].reshape(N, 1), x)

    p1 = sc_spmm(u1, row, col, ew, zeros2)
    t1, u2 = _tc_comb(-1.0, 0.0, p1[0, :N], p1[1, :N], dis, x)
    p2 = sc_spmm(u2, row, col, ew, zeros2)
    _, u3 = _tc_comb(-2.0, -1.0, p2[0, :N], p2[1, :N], dis, x)
    p3 = sc_spmm(u3, row, col, ew, zeros2)

    out = _tc_final(x, t1, p2[0, :N], p2[1, :N], p3[0, :N], p3[1, :N], dis,
                    cheb2_w, cheb2_b.reshape(1, D),
                    cheb3_w, cheb3_b.reshape(1, D),
                    cheb4_w, cheb4_b.reshape(1, D),
                    att_w.T, att_b.reshape(1, 3))
    return (out, edge_scores)


# speedup vs baseline: 7.8539x; 1.2916x over previous
import functools
import math

import jax
import jax.numpy as jnp
from jax import lax
from jax.experimental import pallas as pl
from jax.experimental.pallas import tpu as pltpu
from jax.experimental.pallas import tpu_sc as plsc

N = 10000
E = 320000
D = 128
NPAD = 10240
CHUNK = 128
NC = 2
NS = 16
NW = NC * NS
ROWS_PER_SUB = NPAD // NS
EPAD = 327680
NCHUNKS_P = EPAD // CHUNK
TRIPS_P = NCHUNKS_P // NW


def _gather2_body(a_hbm, b_hbm, pack_hbm, ga_hbm, gb_hbm,
                  pk0, pk1, bufa0, bufa1, bufb0, bufb1,
                  ga0, ga1, gb0, gb1, wa0, wa1, wb0, wb1):
    w = lax.axis_index("s") * NC + lax.axis_index("c")
    pk = (pk0, pk1)
    bufa = (bufa0, bufa1)
    bufb = (bufb0, bufb1)
    gsa = (ga0, ga1)
    gsb = (gb0, gb1)
    wsa = (wa0, wa1)
    wsb = (wb0, wb1)

    def chunk_of(i):
        return w + NW * i

    pltpu.sync_copy(pack_hbm.at[chunk_of(0)], pk0)
    pltpu.async_copy(a_hbm.at[pk0.at[0]], bufa0, ga0)
    pltpu.async_copy(b_hbm.at[pk0.at[1]], bufb0, gb0)

    def body(k, _):
        for p in range(2):
            i = 2 * k + p
            q = 1 - p

            @pl.when(i + 1 < TRIPS_P)
            def _():
                pltpu.sync_copy(pack_hbm.at[chunk_of(i + 1)], pk[q])

                @pl.when(i >= 1)
                def _():
                    pltpu.make_async_copy(
                        bufa[q], ga_hbm.at[pl.ds(0, CHUNK)], wsa[q]).wait()
                    pltpu.make_async_copy(
                        bufb[q], gb_hbm.at[pl.ds(0, CHUNK)], wsb[q]).wait()

                pltpu.async_copy(a_hbm.at[pk[q].at[0]], bufa[q], gsa[q])
                pltpu.async_copy(b_hbm.at[pk[q].at[1]], bufb[q], gsb[q])

            base = chunk_of(i) * CHUNK
            pltpu.make_async_copy(a_hbm.at[pk[p].at[0]], bufa[p], gsa[p]).wait()
            pltpu.make_async_copy(b_hbm.at[pk[p].at[1]], bufb[p], gsb[p]).wait()
            pltpu.async_copy(bufa[p], ga_hbm.at[pl.ds(base, CHUNK)], wsa[p])
            pltpu.async_copy(bufb[p], gb_hbm.at[pl.ds(base, CHUNK)], wsb[p])
        return 0

    lax.fori_loop(0, TRIPS_P // 2, body, 0)
    for p in range(2):
        pltpu.make_async_copy(bufa[p], ga_hbm.at[pl.ds(0, CHUNK)], wsa[p]).wait()
        pltpu.make_async_copy(bufb[p], gb_hbm.at[pl.ds(0, CHUNK)], wsb[p]).wait()


def _deg_body(pack_hbm, ew_hbm, zeros_hbm, dp_hbm,
              riv0, riv1, vv0, vv1, acc, ss0, ss1):
    c = lax.axis_index("c")
    s = lax.axis_index("s")
    w = s * NC + c
    riv = (riv0, riv1)
    vv = (vv0, vv1)
    ss = (ss0, ss1)

    pltpu.sync_copy(zeros_hbm.at[pl.ds(s * ROWS_PER_SUB, ROWS_PER_SUB)],
                    acc.at[pl.ds(s * ROWS_PER_SUB, ROWS_PER_SUB)])
    plsc.subcore_barrier()

    def chunk_of(i):
        return w + NW * i

    pltpu.sync_copy(pack_hbm.at[chunk_of(0), 0], riv0)
    pltpu.sync_copy(ew_hbm.at[pl.ds(chunk_of(0) * CHUNK, CHUNK)], vv0)

    def body(k, _):
        for p in range(2):
            i = 2 * k + p
            q = 1 - p
            pltpu.async_copy(vv[p], acc.at[riv[p]], ss[p], add=True)

            @pl.when(i + 1 < TRIPS_P)
            def _():
                @pl.when(i >= 1)
                def _():
                    pltpu.make_async_copy(vv[q], acc.at[riv[q]], ss[q]).wait()

                pltpu.sync_copy(pack_hbm.at[chunk_of(i + 1), 0], riv[q])
                pltpu.sync_copy(
                    ew_hbm.at[pl.ds(chunk_of(i + 1) * CHUNK, CHUNK)], vv[q])
        return 0

    lax.fori_loop(0, TRIPS_P // 2, body, 0)
    for p in range(2):
        pltpu.make_async_copy(vv[p], acc.at[riv[p]], ss[p]).wait()
    plsc.subcore_barrier()
    pltpu.sync_copy(acc.at[pl.ds(s * ROWS_PER_SUB, ROWS_PER_SUB)],
                    dp_hbm.at[c, pl.ds(s * ROWS_PER_SUB, ROWS_PER_SUB)])


def _spmm_body(u_hbm, pack_hbm, ew_hbm, zeros_hbm, out_hbm,
               pk0, pk1, wv0, wv1, buf0, buf1, acc, gs0, gs1, ss0, ss1):
    c = lax.axis_index("c")
    s = lax.axis_index("s")
    w = s * NC + c
    pk = (pk0, pk1)
    wv = (wv0, wv1)
    buf = (buf0, buf1)
    gs = (gs0, gs1)
    ss = (ss0, ss1)

    pltpu.sync_copy(zeros_hbm.at[pl.ds(s * ROWS_PER_SUB, ROWS_PER_SUB)],
                    acc.at[pl.ds(s * ROWS_PER_SUB, ROWS_PER_SUB)])
    plsc.subcore_barrier()

    def chunk_of(i):
        return w + NW * i

    pltpu.sync_copy(pack_hbm.at[chunk_of(0)], pk0)
    pltpu.sync_copy(ew_hbm.at[pl.ds(chunk_of(0) * CHUNK, CHUNK)], wv0)
    pltpu.async_copy(u_hbm.at[pk0.at[0]], buf0, gs0)

    def body(k, _):
        for p in range(2):
            i = 2 * k + p
            q = 1 - p

            @pl.when(i + 1 < TRIPS_P)
            def _():
                pltpu.sync_copy(pack_hbm.at[chunk_of(i + 1)], pk[q])
                pltpu.sync_copy(
                    ew_hbm.at[pl.ds(chunk_of(i + 1) * CHUNK, CHUNK)], wv[q])

                @pl.when(i >= 1)
                def _():
                    pltpu.make_async_copy(buf[q], acc.at[pk[q].at[1]], ss[q]).wait()

                pltpu.async_copy(u_hbm.at[pk[q].at[0]], buf[q], gs[q])

            pltpu.make_async_copy(u_hbm.at[pk[p].at[0]], buf[p], gs[p]).wait()

            def scale(g, _):
                wvec = wv[p][pl.ds(g * 16, 16)]
                for l in range(16):
                    we = wvec[l]
                    e = g * 16 + l
                    for j in range(D // 16):
                        sl = pl.ds(j * 16, 16)
                        buf[p][e, sl] = buf[p][e, sl] * we
                return 0

            lax.fori_loop(0, CHUNK // 16, scale, 0)
            pltpu.async_copy(buf[p], acc.at[pk[p].at[1]], ss[p], add=True)
        return 0

    lax.fori_loop(0, TRIPS_P // 2, body, 0)
    for p in range(2):
        pltpu.make_async_copy(buf[p], acc.at[pk[p].at[1]], ss[p]).wait()
    plsc.subcore_barrier()
    pltpu.sync_copy(acc.at[pl.ds(s * ROWS_PER_SUB, ROWS_PER_SUB)],
                    out_hbm.at[c, pl.ds(s * ROWS_PER_SUB, ROWS_PER_SUB)])


@functools.lru_cache(maxsize=1)
def _sc_kernels():
    mesh = plsc.VectorSubcoreMesh(core_axis_name="c", subcore_axis_name="s",
                                  num_cores=NC, num_subcores=NS)
    gather2 = pl.kernel(
        _gather2_body,
        out_type=(
            jax.ShapeDtypeStruct((EPAD, D), jnp.float32),
            jax.ShapeDtypeStruct((EPAD, D), jnp.float32),
        ),
        mesh=mesh,
        scratch_types=[
            pltpu.VMEM((2, CHUNK), jnp.int32),
            pltpu.VMEM((2, CHUNK), jnp.int32),
            pltpu.VMEM((CHUNK, D), jnp.float32),
            pltpu.VMEM((CHUNK, D), jnp.float32),
            pltpu.VMEM((CHUNK, D), jnp.float32),
            pltpu.VMEM((CHUNK, D), jnp.float32),
        ] + [pltpu.SemaphoreType.DMA] * 8,
    )
    deg = pl.kernel(
        _deg_body,
        out_type=jax.ShapeDtypeStruct((NC, NPAD), jnp.float32),
        mesh=mesh,
        scratch_types=[
            pltpu.VMEM((CHUNK,), jnp.int32),
            pltpu.VMEM((CHUNK,), jnp.int32),
            pltpu.VMEM((CHUNK,), jnp.float32),
            pltpu.VMEM((CHUNK,), jnp.float32),
            pltpu.VMEM_SHARED((NPAD,), jnp.float32),
        ] + [pltpu.SemaphoreType.DMA] * 2,
    )
    spmm = pl.kernel(
        _spmm_body,
        out_type=jax.ShapeDtypeStruct((NC, NPAD, D), jnp.float32),
        mesh=mesh,
        scratch_types=[
            pltpu.VMEM((2, CHUNK), jnp.int32),
            pltpu.VMEM((2, CHUNK), jnp.int32),
            pltpu.VMEM((CHUNK,), jnp.float32),
            pltpu.VMEM((CHUNK,), jnp.float32),
            pltpu.VMEM((CHUNK, D), jnp.float32),
            pltpu.VMEM((CHUNK, D), jnp.float32),
            pltpu.VMEM_SHARED((NPAD, D), jnp.float32),
        ] + [pltpu.SemaphoreType.DMA] * 4,
    )
    return gather2, deg, spmm


_NBLK = 1000
_EBLK = 2000


def _rsqrt_precise(v):
    r = lax.rsqrt(v)
    return r * (1.5 - 0.5 * v * r * r)


def _tc_ab_body(x_ref, w1a_ref, w1b_ref, b1_ref, a_ref, b_ref):
    x = x_ref[...]
    a_ref[...] = jnp.dot(x, w1a_ref[...], preferred_element_type=jnp.float32,
                         precision=lax.Precision.HIGHEST) + b1_ref[...]
    b_ref[...] = jnp.dot(x, w1b_ref[...], preferred_element_type=jnp.float32,
                         precision=lax.Precision.HIGHEST)


def _tc_ab(x, w1a, w1b, b1):
    grid = (N // _NBLK,)
    return pl.pallas_call(
        _tc_ab_body,
        grid=grid,
        in_specs=[
            pl.BlockSpec((_NBLK, D), lambda i: (i, 0)),
            pl.BlockSpec((D, D), lambda i: (0, 0)),
            pl.BlockSpec((D, D), lambda i: (0, 0)),
            pl.BlockSpec((1, D), lambda i: (0, 0)),
        ],
        out_specs=[
            pl.BlockSpec((_NBLK, D), lambda i: (i, 0)),
            pl.BlockSpec((_NBLK, D), lambda i: (i, 0)),
        ],
        out_shape=[
            jax.ShapeDtypeStruct((N, D), jnp.float32),
            jax.ShapeDtypeStruct((N, D), jnp.float32),
        ],
    )(x, w1a, w1b, b1)


def _tc_edge_body(ga_ref, gb_ref, g_ref, b_ref, w2_ref, b2_ref, s_ref):
    h = ga_ref[...] + gb_ref[...]
    mu = jnp.mean(h, axis=-1, keepdims=True)
    hc = h - mu
    var = jnp.mean(hc * hc, axis=-1, keepdims=True)
    hn = hc * _rsqrt_precise(var + 1e-5) * g_ref[...] + b_ref[...]
    hg = 0.5 * hn * (1.0 + lax.erf(hn * (1.0 / math.sqrt(2.0))))
    s = jnp.sum(hg * w2_ref[...], axis=-1, keepdims=True) + b2_ref[...]
    s_ref[...] = jax.nn.sigmoid(s)


def _tc_edge(ga, gb, ln_g, ln_b, w2row, b2):
    grid = (E // _EBLK,)
    return pl.pallas_call(
        _tc_edge_body,
        grid=grid,
        in_specs=[
            pl.BlockSpec((_EBLK, D), lambda i: (i, 0)),
            pl.BlockSpec((_EBLK, D), lambda i: (i, 0)),
            pl.BlockSpec((1, D), lambda i: (0, 0)),
            pl.BlockSpec((1, D), lambda i: (0, 0)),
            pl.BlockSpec((1, D), lambda i: (0, 0)),
            pl.BlockSpec((1, 1), lambda i: (0, 0)),
        ],
        out_specs=pl.BlockSpec((_EBLK, 1), lambda i: (i, 0)),
        out_shape=jax.ShapeDtypeStruct((E, 1), jnp.float32),
    )(ga, gb, ln_g, ln_b, w2row, b2)


def _tc_deg_body(dp0_ref, dp1_ref, x_ref, dis_ref, u_ref):
    deg = dp0_ref[...] + dp1_ref[...]
    pos = deg > 0
    dis = jnp.where(pos, _rsqrt_precise(jnp.where(pos, deg, 1.0)), 0.0)
    dis_ref[...] = dis
    u_ref[...] = x_ref[...] * dis


def _tc_deg(dp0, dp1, x):
    grid = (N // _NBLK,)
    return pl.pallas_call(
        _tc_deg_body,
        grid=grid,
        in_specs=[
            pl.BlockSpec((_NBLK, 1), lambda i: (i, 0)),
            pl.BlockSpec((_NBLK, 1), lambda i: (i, 0)),
            pl.BlockSpec((_NBLK, D), lambda i: (i, 0)),
        ],
        out_specs=[
            pl.BlockSpec((_NBLK, 1), lambda i: (i, 0)),
            pl.BlockSpec((_NBLK, D), lambda i: (i, 0)),
        ],
        out_shape=[
            jax.ShapeDtypeStruct((N, 1), jnp.float32),
            jax.ShapeDtypeStruct((N, D), jnp.float32),
        ],
    )(dp0, dp1, x)


def _tc_comb_body(alpha, beta, p0_ref, p1_ref, dis_ref, prev_ref, t_ref, u_ref):
    dis = dis_ref[...]
    t = alpha * dis * (p0_ref[...] + p1_ref[...]) + beta * prev_ref[...]
    t_ref[...] = t
    u_ref[...] = dis * t


def _tc_comb(alpha, beta, p0, p1, dis, prev):
    grid = (N // _NBLK,)
    return pl.pallas_call(
        functools.partial(_tc_comb_body, alpha, beta),
        grid=grid,
        in_specs=[
            pl.BlockSpec((_NBLK, D), lambda i: (i, 0)),
            pl.BlockSpec((_NBLK, D), lambda i: (i, 0)),
            pl.BlockSpec((_NBLK, 1), lambda i: (i, 0)),
            pl.BlockSpec((_NBLK, D), lambda i: (i, 0)),
        ],
        out_specs=[
            pl.BlockSpec((_NBLK, D), lambda i: (i, 0)),
            pl.BlockSpec((_NBLK, D), lambda i: (i, 0)),
        ],
        out_shape=[
            jax.ShapeDtypeStruct((N, D), jnp.float32),
            jax.ShapeDtypeStruct((N, D), jnp.float32),
        ],
    )(p0, p1, dis, prev)


def _tc_final_body(x_ref, t1_ref, p2a_ref, p2b_ref, p3a_ref, p3b_ref,
                   dis_ref, w2_ref, b2_ref, w3_ref, b3_ref, w4_ref, b4_ref,
                   awt_ref, ab_ref, out_ref):
    dis = dis_ref[...]
    t0 = x_ref[...]
    t1 = t1_ref[...]
    t2 = -2.0 * dis * (p2a_ref[...] + p2b_ref[...]) - t0
    t3 = -2.0 * dis * (p3a_ref[...] + p3b_ref[...]) - t1

    def mm(a, w):
        return jnp.dot(a, w, preferred_element_type=jnp.float32,
                       precision=lax.Precision.HIGHEST)

    f2 = mm(t0, w2_ref[0]) + mm(t1, w2_ref[1]) + b2_ref[...]
    f3 = mm(t0, w3_ref[0]) + mm(t1, w3_ref[1]) + mm(t2, w3_ref[2]) + b3_ref[...]
    f4 = (mm(t0, w4_ref[0]) + mm(t1, w4_ref[1]) + mm(t2, w4_ref[2])
          + mm(t3, w4_ref[3]) + b4_ref[...])

    awt = awt_ref[...]
    fs = (f2, f3, f4)
    s = []
    for j in range(3):
        acc = ab_ref[0, j]
        tot = None
        for k in range(3):
            term = fs[k] * awt[j:j + 1, k * D:(k + 1) * D]
            tot = term if tot is None else tot + term
        s.append(jnp.sum(tot, axis=-1, keepdims=True) + acc)
    m = jnp.maximum(jnp.maximum(s[0], s[1]), s[2])
    e0 = jnp.exp(s[0] - m)
    e1 = jnp.exp(s[1] - m)
    e2 = jnp.exp(s[2] - m)
    out_ref[...] = (f2 * e0 + f3 * e1 + f4 * e2) / (e0 + e1 + e2)


def _tc_final(x, t1, p2a, p2b, p3a, p3b, dis, w2, b2, w3, b3, w4, b4, awt, ab):
    grid = (N // _NBLK,)
    full = lambda shape: pl.BlockSpec(shape, lambda i: tuple(0 for _ in shape))
    blk = pl.BlockSpec((_NBLK, D), lambda i: (i, 0))
    return pl.pallas_call(
        _tc_final_body,
        grid=grid,
        in_specs=[
            blk, blk, blk, blk, blk, blk,
            pl.BlockSpec((_NBLK, 1), lambda i: (i, 0)),
            full((2, D, D)), full((1, D)),
            full((3, D, D)), full((1, D)),
            full((4, D, D)), full((1, D)),
            full((3, 3 * D)),
            pl.BlockSpec((1, 3), lambda i: (0, 0), memory_space=pltpu.SMEM),
        ],
        out_specs=blk,
        out_shape=jax.ShapeDtypeStruct((N, D), jnp.float32),
    )(x, t1, p2a, p2b, p3a, p3b, dis, w2, b2, w3, b3, w4, b4, awt, ab)


def kernel(x, edge_index, ss_w1, ss_b1, ss_ln_g, ss_ln_b, ss_w2, ss_b2,
           cheb2_w, cheb2_b, cheb3_w, cheb3_b, cheb4_w, cheb4_b, att_w, att_b):
    row = edge_index[0]
    col = edge_index[1]
    sc_gather2, sc_deg, sc_spmm = _sc_kernels()

    fill = (jnp.arange(EPAD - E, dtype=jnp.int32) * 97) % N
    row_p = jnp.concatenate([row, fill])
    col_p = jnp.concatenate([col, fill])
    pack = jnp.stack([row_p.reshape(NCHUNKS_P, CHUNK),
                      col_p.reshape(NCHUNKS_P, CHUNK)], axis=1)

    a_nodes, b_nodes = _tc_ab(x, ss_w1[:D], ss_w1[D:], ss_b1.reshape(1, D))
    ga, gb = sc_gather2(a_nodes, b_nodes, pack)
    edge_scores = _tc_edge(ga[:E], gb[:E], ss_ln_g.reshape(1, D),
                           ss_ln_b.reshape(1, D), ss_w2.reshape(1, D),
                           ss_b2.reshape(1, 1))
    ew = edge_scores.reshape(E)
    ew_p = jnp.concatenate([ew, jnp.zeros((EPAD - E,), jnp.float32)])

    zeros1 = jnp.zeros((NPAD,), jnp.float32)
    zeros2 = jnp.zeros((NPAD, D), jnp.float32)

    dp = sc_deg(pack, ew_p, zeros1)
    dis, u1 = _tc_deg(dp[0, :N].reshape(N, 1), dp[1, :N].reshape(N, 1), x)

    p1 = sc_spmm(u1, pack, ew_p, zeros2)
    t1, u2 = _tc_comb(-1.0, 0.0, p1[0, :N], p1[1, :N], dis, x)
    p2 = sc_spmm(u2, pack, ew_p, zeros2)
    _, u3 = _tc_comb(-2.0, -1.0, p2[0, :N], p2[1, :N], dis, x)
    p3 = sc_spmm(u3, pack, ew_p, zeros2)

    out = _tc_final(x, t1, p2[0, :N], p2[1, :N], p3[0, :N], p3[1, :N], dis,
                    cheb2_w, cheb2_b.reshape(1, D),
                    cheb3_w, cheb3_b.reshape(1, D),
                    cheb4_w, cheb4_b.reshape(1, D),
                    att_w.T, att_b.reshape(1, 3))
    return (out, edge_scores)


# speedup vs baseline: 8.8896x; 1.1319x over previous
import functools
import math

import jax
import jax.numpy as jnp
from jax import lax
from jax.experimental import pallas as pl
from jax.experimental.pallas import tpu as pltpu
from jax.experimental.pallas import tpu_sc as plsc

N = 10000
E = 320000
D = 128
NPAD = 10240
CHUNK = 128
NC = 2
NS = 16
NW = NC * NS
ROWS_PER_SUB = NPAD // NS
EPAD = 327680
NCHUNKS_P = EPAD // CHUNK
TRIPS_P = NCHUNKS_P // NW


def _gatherh_body(a_hbm, b_hbm, pack_hbm, h_hbm,
                  pk0, pk1, bufa0, bufa1, bufb0, bufb1,
                  ga0, ga1, gb0, gb1, wa0, wa1):
    w = lax.axis_index("s") * NC + lax.axis_index("c")
    pk = (pk0, pk1)
    bufa = (bufa0, bufa1)
    bufb = (bufb0, bufb1)
    gsa = (ga0, ga1)
    gsb = (gb0, gb1)
    wsa = (wa0, wa1)

    def chunk_of(i):
        return w + NW * i

    pltpu.sync_copy(pack_hbm.at[chunk_of(0)], pk0)
    pltpu.async_copy(a_hbm.at[pk0.at[0]], bufa0, ga0)
    pltpu.async_copy(b_hbm.at[pk0.at[1]], bufb0, gb0)

    def body(k, _):
        for p in range(2):
            i = 2 * k + p
            q = 1 - p

            @pl.when(i + 1 < TRIPS_P)
            def _():
                pltpu.sync_copy(pack_hbm.at[chunk_of(i + 1)], pk[q])

                @pl.when(i >= 1)
                def _():
                    pltpu.make_async_copy(
                        bufa[q], h_hbm.at[pl.ds(0, CHUNK)], wsa[q]).wait()

                pltpu.async_copy(a_hbm.at[pk[q].at[0]], bufa[q], gsa[q])
                pltpu.async_copy(b_hbm.at[pk[q].at[1]], bufb[q], gsb[q])

            base = chunk_of(i) * CHUNK
            pltpu.make_async_copy(a_hbm.at[pk[p].at[0]], bufa[p], gsa[p]).wait()
            pltpu.make_async_copy(b_hbm.at[pk[p].at[1]], bufb[p], gsb[p]).wait()

            def addrows(g, _):
                for l in range(16):
                    e = g * 16 + l
                    for j in range(D // 16):
                        sl = pl.ds(j * 16, 16)
                        bufa[p][e, sl] = bufa[p][e, sl] + bufb[p][e, sl]
                return 0

            lax.fori_loop(0, CHUNK // 16, addrows, 0)
            pltpu.async_copy(bufa[p], h_hbm.at[pl.ds(base, CHUNK)], wsa[p])
        return 0

    lax.fori_loop(0, TRIPS_P // 2, body, 0)
    for p in range(2):
        pltpu.make_async_copy(bufa[p], h_hbm.at[pl.ds(0, CHUNK)], wsa[p]).wait()


def _deg_body(pack_hbm, ew_hbm, zeros_hbm, dp_hbm,
              riv0, riv1, vv0, vv1, acc, ss0, ss1):
    c = lax.axis_index("c")
    s = lax.axis_index("s")
    w = s * NC + c
    riv = (riv0, riv1)
    vv = (vv0, vv1)
    ss = (ss0, ss1)

    pltpu.sync_copy(zeros_hbm.at[pl.ds(s * ROWS_PER_SUB, ROWS_PER_SUB)],
                    acc.at[pl.ds(s * ROWS_PER_SUB, ROWS_PER_SUB)])
    plsc.subcore_barrier()

    def chunk_of(i):
        return w + NW * i

    pltpu.sync_copy(pack_hbm.at[chunk_of(0), 0], riv0)
    pltpu.sync_copy(ew_hbm.at[pl.ds(chunk_of(0) * CHUNK, CHUNK)], vv0)

    def body(k, _):
        for p in range(2):
            i = 2 * k + p
            q = 1 - p
            pltpu.async_copy(vv[p], acc.at[riv[p]], ss[p], add=True)

            @pl.when(i + 1 < TRIPS_P)
            def _():
                @pl.when(i >= 1)
                def _():
                    pltpu.make_async_copy(vv[q], acc.at[riv[q]], ss[q]).wait()

                pltpu.sync_copy(pack_hbm.at[chunk_of(i + 1), 0], riv[q])
                pltpu.sync_copy(
                    ew_hbm.at[pl.ds(chunk_of(i + 1) * CHUNK, CHUNK)], vv[q])
        return 0

    lax.fori_loop(0, TRIPS_P // 2, body, 0)
    for p in range(2):
        pltpu.make_async_copy(vv[p], acc.at[riv[p]], ss[p]).wait()
    plsc.subcore_barrier()
    pltpu.sync_copy(acc.at[pl.ds(s * ROWS_PER_SUB, ROWS_PER_SUB)],
                    dp_hbm.at[c, pl.ds(s * ROWS_PER_SUB, ROWS_PER_SUB)])


def _spmm_body(u_hbm, pack_hbm, ew_hbm, zeros_hbm, out_hbm,
               pk0, pk1, wv0, wv1, buf0, buf1, acc, gs0, gs1, ss0, ss1):
    c = lax.axis_index("c")
    s = lax.axis_index("s")
    w = s * NC + c
    pk = (pk0, pk1)
    wv = (wv0, wv1)
    buf = (buf0, buf1)
    gs = (gs0, gs1)
    ss = (ss0, ss1)

    pltpu.sync_copy(zeros_hbm.at[pl.ds(s * ROWS_PER_SUB, ROWS_PER_SUB)],
                    acc.at[pl.ds(s * ROWS_PER_SUB, ROWS_PER_SUB)])
    plsc.subcore_barrier()

    def chunk_of(i):
        return w + NW * i

    pltpu.sync_copy(pack_hbm.at[chunk_of(0)], pk0)
    pltpu.sync_copy(ew_hbm.at[pl.ds(chunk_of(0) * CHUNK, CHUNK)], wv0)
    pltpu.async_copy(u_hbm.at[pk0.at[0]], buf0, gs0)

    def body(k, _):
        for p in range(2):
            i = 2 * k + p
            q = 1 - p

            @pl.when(i + 1 < TRIPS_P)
            def _():
                pltpu.sync_copy(pack_hbm.at[chunk_of(i + 1)], pk[q])
                pltpu.sync_copy(
                    ew_hbm.at[pl.ds(chunk_of(i + 1) * CHUNK, CHUNK)], wv[q])

                @pl.when(i >= 1)
                def _():
                    pltpu.make_async_copy(buf[q], acc.at[pk[q].at[1]], ss[q]).wait()

                pltpu.async_copy(u_hbm.at[pk[q].at[0]], buf[q], gs[q])

            pltpu.make_async_copy(u_hbm.at[pk[p].at[0]], buf[p], gs[p]).wait()

            def scale(g, _):
                wvec = wv[p][pl.ds(g * 16, 16)]
                for l in range(16):
                    we = wvec[l]
                    e = g * 16 + l
                    for j in range(D // 16):
                        sl = pl.ds(j * 16, 16)
                        buf[p][e, sl] = buf[p][e, sl] * we
                return 0

            lax.fori_loop(0, CHUNK // 16, scale, 0)
            pltpu.async_copy(buf[p], acc.at[pk[p].at[1]], ss[p], add=True)
        return 0

    lax.fori_loop(0, TRIPS_P // 2, body, 0)
    for p in range(2):
        pltpu.make_async_copy(buf[p], acc.at[pk[p].at[1]], ss[p]).wait()
    plsc.subcore_barrier()
    pltpu.sync_copy(acc.at[pl.ds(s * ROWS_PER_SUB, ROWS_PER_SUB)],
                    out_hbm.at[c, pl.ds(s * ROWS_PER_SUB, ROWS_PER_SUB)])


@functools.lru_cache(maxsize=1)
def _sc_kernels():
    mesh = plsc.VectorSubcoreMesh(core_axis_name="c", subcore_axis_name="s",
                                  num_cores=NC, num_subcores=NS)
    gatherh = pl.kernel(
        _gatherh_body,
        out_type=jax.ShapeDtypeStruct((EPAD, D), jnp.float32),
        mesh=mesh,
        scratch_types=[
            pltpu.VMEM((2, CHUNK), jnp.int32),
            pltpu.VMEM((2, CHUNK), jnp.int32),
            pltpu.VMEM((CHUNK, D), jnp.float32),
            pltpu.VMEM((CHUNK, D), jnp.float32),
            pltpu.VMEM((CHUNK, D), jnp.float32),
            pltpu.VMEM((CHUNK, D), jnp.float32),
        ] + [pltpu.SemaphoreType.DMA] * 6,
    )
    deg = pl.kernel(
        _deg_body,
        out_type=jax.ShapeDtypeStruct((NC, NPAD), jnp.float32),
        mesh=mesh,
        scratch_types=[
            pltpu.VMEM((CHUNK,), jnp.int32),
            pltpu.VMEM((CHUNK,), jnp.int32),
            pltpu.VMEM((CHUNK,), jnp.float32),
            pltpu.VMEM((CHUNK,), jnp.float32),
            pltpu.VMEM_SHARED((NPAD,), jnp.float32),
        ] + [pltpu.SemaphoreType.DMA] * 2,
    )
    spmm = pl.kernel(
        _spmm_body,
        out_type=jax.ShapeDtypeStruct((NC, NPAD, D), jnp.float32),
        mesh=mesh,
        scratch_types=[
            pltpu.VMEM((2, CHUNK), jnp.int32),
            pltpu.VMEM((2, CHUNK), jnp.int32),
            pltpu.VMEM((CHUNK,), jnp.float32),
            pltpu.VMEM((CHUNK,), jnp.float32),
            pltpu.VMEM((CHUNK, D), jnp.float32),
            pltpu.VMEM((CHUNK, D), jnp.float32),
            pltpu.VMEM_SHARED((NPAD, D), jnp.float32),
        ] + [pltpu.SemaphoreType.DMA] * 4,
    )
    return gatherh, deg, spmm


_NBLK = 1000
_EBLK = 2000


def _rsqrt_precise(v):
    r = lax.rsqrt(v)
    return r * (1.5 - 0.5 * v * r * r)


def _tc_ab_body(x_ref, w1a_ref, w1b_ref, b1_ref, a_ref, b_ref):
    x = x_ref[...]
    a_ref[...] = jnp.dot(x, w1a_ref[...], preferred_element_type=jnp.float32,
                         precision=lax.Precision.HIGHEST) + b1_ref[...]
    b_ref[...] = jnp.dot(x, w1b_ref[...], preferred_element_type=jnp.float32,
                         precision=lax.Precision.HIGHEST)


def _tc_ab(x, w1a, w1b, b1):
    grid = (N // _NBLK,)
    return pl.pallas_call(
        _tc_ab_body,
        grid=grid,
        in_specs=[
            pl.BlockSpec((_NBLK, D), lambda i: (i, 0)),
            pl.BlockSpec((D, D), lambda i: (0, 0)),
            pl.BlockSpec((D, D), lambda i: (0, 0)),
            pl.BlockSpec((1, D), lambda i: (0, 0)),
        ],
        out_specs=[
            pl.BlockSpec((_NBLK, D), lambda i: (i, 0)),
            pl.BlockSpec((_NBLK, D), lambda i: (i, 0)),
        ],
        out_shape=[
            jax.ShapeDtypeStruct((N, D), jnp.float32),
            jax.ShapeDtypeStruct((N, D), jnp.float32),
        ],
    )(x, w1a, w1b, b1)


def _tc_edge_body(h_ref, g_ref, b_ref, w2_ref, b2_ref, s_ref):
    h = h_ref[...]
    mu = jnp.mean(h, axis=-1, keepdims=True)
    hc = h - mu
    var = jnp.mean(hc * hc, axis=-1, keepdims=True)
    hn = hc * _rsqrt_precise(var + 1e-5) * g_ref[...] + b_ref[...]
    hg = 0.5 * hn * (1.0 + lax.erf(hn * (1.0 / math.sqrt(2.0))))
    s = jnp.sum(hg * w2_ref[...], axis=-1, keepdims=True) + b2_ref[...]
    s_ref[...] = jax.nn.sigmoid(s)


def _tc_edge(h, ln_g, ln_b, w2row, b2):
    grid = (E // _EBLK,)
    return pl.pallas_call(
        _tc_edge_body,
        grid=grid,
        in_specs=[
            pl.BlockSpec((_EBLK, D), lambda i: (i, 0)),
            pl.BlockSpec((1, D), lambda i: (0, 0)),
            pl.BlockSpec((1, D), lambda i: (0, 0)),
            pl.BlockSpec((1, D), lambda i: (0, 0)),
            pl.BlockSpec((1, 1), lambda i: (0, 0)),
        ],
        out_specs=pl.BlockSpec((_EBLK, 1), lambda i: (i, 0)),
        out_shape=jax.ShapeDtypeStruct((E, 1), jnp.float32),
    )(h, ln_g, ln_b, w2row, b2)


def _tc_deg_body(dp0_ref, dp1_ref, x_ref, dis_ref, u_ref):
    deg = dp0_ref[...] + dp1_ref[...]
    pos = deg > 0
    dis = jnp.where(pos, _rsqrt_precise(jnp.where(pos, deg, 1.0)), 0.0)
    dis_ref[...] = dis
    u_ref[...] = x_ref[...] * dis


def _tc_deg(dp0, dp1, x):
    grid = (N // _NBLK,)
    return pl.pallas_call(
        _tc_deg_body,
        grid=grid,
        in_specs=[
            pl.BlockSpec((_NBLK, 1), lambda i: (i, 0)),
            pl.BlockSpec((_NBLK, 1), lambda i: (i, 0)),
            pl.BlockSpec((_NBLK, D), lambda i: (i, 0)),
        ],
        out_specs=[
            pl.BlockSpec((_NBLK, 1), lambda i: (i, 0)),
            pl.BlockSpec((_NBLK, D), lambda i: (i, 0)),
        ],
        out_shape=[
            jax.ShapeDtypeStruct((N, 1), jnp.float32),
            jax.ShapeDtypeStruct((N, D), jnp.float32),
        ],
    )(dp0, dp1, x)


def _tc_comb_body(alpha, beta, p_ref, dis_ref, prev_ref, t_ref, u_ref):
    dis = dis_ref[...]
    t = alpha * dis * (p_ref[0] + p_ref[1]) + beta * prev_ref[...]
    t_ref[...] = t
    u_ref[...] = dis * t


def _tc_comb(alpha, beta, p, dis, prev):
    grid = (N // _NBLK,)
    return pl.pallas_call(
        functools.partial(_tc_comb_body, alpha, beta),
        grid=grid,
        in_specs=[
            pl.BlockSpec((2, _NBLK, D), lambda i: (0, i, 0)),
            pl.BlockSpec((_NBLK, 1), lambda i: (i, 0)),
            pl.BlockSpec((_NBLK, D), lambda i: (i, 0)),
        ],
        out_specs=[
            pl.BlockSpec((_NBLK, D), lambda i: (i, 0)),
            pl.BlockSpec((_NBLK, D), lambda i: (i, 0)),
        ],
        out_shape=[
            jax.ShapeDtypeStruct((N, D), jnp.float32),
            jax.ShapeDtypeStruct((N, D), jnp.float32),
        ],
    )(p, dis, prev)


def _tc_final_body(x_ref, t1_ref, p2_ref, p3_ref,
                   dis_ref, w2_ref, b2_ref, w3_ref, b3_ref, w4_ref, b4_ref,
                   awt_ref, ab_ref, out_ref):
    dis = dis_ref[...]
    t0 = x_ref[...]
    t1 = t1_ref[...]
    t2 = -2.0 * dis * (p2_ref[0] + p2_ref[1]) - t0
    t3 = -2.0 * dis * (p3_ref[0] + p3_ref[1]) - t1

    def mm(a, w):
        return jnp.dot(a, w, preferred_element_type=jnp.float32,
                       precision=lax.Precision.HIGHEST)

    f2 = mm(t0, w2_ref[0]) + mm(t1, w2_ref[1]) + b2_ref[...]
    f3 = mm(t0, w3_ref[0]) + mm(t1, w3_ref[1]) + mm(t2, w3_ref[2]) + b3_ref[...]
    f4 = (mm(t0, w4_ref[0]) + mm(t1, w4_ref[1]) + mm(t2, w4_ref[2])
          + mm(t3, w4_ref[3]) + b4_ref[...])

    awt = awt_ref[...]
    fs = (f2, f3, f4)
    s = []
    for j in range(3):
        acc = ab_ref[0, j]
        tot = None
        for k in range(3):
            term = fs[k] * awt[j:j + 1, k * D:(k + 1) * D]
            tot = term if tot is None else tot + term
        s.append(jnp.sum(tot, axis=-1, keepdims=True) + acc)
    m = jnp.maximum(jnp.maximum(s[0], s[1]), s[2])
    e0 = jnp.exp(s[0] - m)
    e1 = jnp.exp(s[1] - m)
    e2 = jnp.exp(s[2] - m)
    out_ref[...] = (f2 * e0 + f3 * e1 + f4 * e2) / (e0 + e1 + e2)


def _tc_final(x, t1, p2, p3, dis, w2, b2, w3, b3, w4, b4, awt, ab):
    grid = (N // _NBLK,)
    full = lambda shape: pl.BlockSpec(shape, lambda i: tuple(0 for _ in shape))
    blk = pl.BlockSpec((_NBLK, D), lambda i: (i, 0))
    pblk = pl.BlockSpec((2, _NBLK, D), lambda i: (0, i, 0))
    return pl.pallas_call(
        _tc_final_body,
        grid=grid,
        in_specs=[
            blk, blk, pblk, pblk,
            pl.BlockSpec((_NBLK, 1), lambda i: (i, 0)),
            full((2, D, D)), full((1, D)),
            full((3, D, D)), full((1, D)),
            full((4, D, D)), full((1, D)),
            full((3, 3 * D)),
            pl.BlockSpec((1, 3), lambda i: (0, 0), memory_space=pltpu.SMEM),
        ],
        out_specs=blk,
        out_shape=jax.ShapeDtypeStruct((N, D), jnp.float32),
    )(x, t1, p2, p3, dis, w2, b2, w3, b3, w4, b4, awt, ab)


def kernel(x, edge_index, ss_w1, ss_b1, ss_ln_g, ss_ln_b, ss_w2, ss_b2,
           cheb2_w, cheb2_b, cheb3_w, cheb3_b, cheb4_w, cheb4_b, att_w, att_b):
    row = edge_index[0]
    col = edge_index[1]
    sc_gatherh, sc_deg, sc_spmm = _sc_kernels()

    fill = (jnp.arange(EPAD - E, dtype=jnp.int32) * 97) % N
    row_p = jnp.concatenate([row, fill])
    col_p = jnp.concatenate([col, fill])
    pack = jnp.stack([row_p.reshape(NCHUNKS_P, CHUNK),
                      col_p.reshape(NCHUNKS_P, CHUNK)], axis=1)

    a_nodes, b_nodes = _tc_ab(x, ss_w1[:D], ss_w1[D:], ss_b1.reshape(1, D))
    h_edges = sc_gatherh(a_nodes, b_nodes, pack)
    edge_scores = _tc_edge(h_edges, ss_ln_g.reshape(1, D),
                           ss_ln_b.reshape(1, D), ss_w2.reshape(1, D),
                           ss_b2.reshape(1, 1))
    ew = edge_scores.reshape(E)
    ew_p = jnp.concatenate([ew, jnp.zeros((EPAD - E,), jnp.float32)])

    zeros1 = jnp.zeros((NPAD,), jnp.float32)
    zeros2 = jnp.zeros((NPAD, D), jnp.float32)

    dp = sc_deg(pack, ew_p, zeros1)
    dis, u1 = _tc_deg(dp[0, :N].reshape(N, 1), dp[1, :N].reshape(N, 1), x)

    p1 = sc_spmm(u1, pack, ew_p, zeros2)
    t1, u2 = _tc_comb(-1.0, 0.0, p1, dis, x)
    p2 = sc_spmm(u2, pack, ew_p, zeros2)
    _, u3 = _tc_comb(-2.0, -1.0, p2, dis, x)
    p3 = sc_spmm(u3, pack, ew_p, zeros2)

    out = _tc_final(x, t1, p2, p3, dis,
                    cheb2_w, cheb2_b.reshape(1, D),
                    cheb3_w, cheb3_b.reshape(1, D),
                    cheb4_w, cheb4_b.reshape(1, D),
                    att_w.T, att_b.reshape(1, 3))
    return (out, edge_scores)


# speedup vs baseline: 10.9232x; 1.2288x over previous
import functools
import math

import jax
import jax.numpy as jnp
from jax import lax
from jax.experimental import pallas as pl
from jax.experimental.pallas import tpu as pltpu
from jax.experimental.pallas import tpu_sc as plsc

N = 10000
E = 320000
D = 128
NPAD = 10240
CHUNK = 128
NC = 2
NS = 16
NW = NC * NS
ROWS_PER_SUB = NPAD // NS
EPAD = 327680
NCHUNKS_P = EPAD // CHUNK
TRIPS_P = NCHUNKS_P // NW


def _gatherh_body(a_hbm, b_hbm, packw_hbm, h_hbm,
                  pkall, bufa0, bufa1, bufb0, bufb1,
                  ga0, ga1, gb0, gb1, wa0, wa1):
    w = lax.axis_index("s") * NC + lax.axis_index("c")
    bufa = (bufa0, bufa1)
    bufb = (bufb0, bufb1)
    gsa = (ga0, ga1)
    gsb = (gb0, gb1)
    wsa = (wa0, wa1)

    pltpu.sync_copy(packw_hbm.at[w], pkall)
    pltpu.async_copy(a_hbm.at[pkall.at[0, 0]], bufa0, ga0)
    pltpu.async_copy(b_hbm.at[pkall.at[0, 1]], bufb0, gb0)

    def body(k, _):
        for p in range(2):
            i = 2 * k + p
            q = 1 - p

            @pl.when(i + 1 < TRIPS_P)
            def _():
                @pl.when(i >= 1)
                def _():
                    pltpu.make_async_copy(
                        bufa[q], h_hbm.at[pl.ds(0, CHUNK)], wsa[q]).wait()

                pltpu.async_copy(a_hbm.at[pkall.at[i + 1, 0]], bufa[q], gsa[q])
                pltpu.async_copy(b_hbm.at[pkall.at[i + 1, 1]], bufb[q], gsb[q])

            base = (w + NW * i) * CHUNK
            pltpu.make_async_copy(a_hbm.at[pkall.at[i, 0]], bufa[p], gsa[p]).wait()
            pltpu.make_async_copy(b_hbm.at[pkall.at[i, 1]], bufb[p], gsb[p]).wait()

            def addrows(g, _):
                for l in range(16):
                    e = g * 16 + l
                    for j in range(D // 16):
                        sl = pl.ds(j * 16, 16)
                        plsc.addupdate(bufa[p].at[e, sl], bufb[p][e, sl])
                return 0

            lax.fori_loop(0, CHUNK // 16, addrows, 0)
            pltpu.async_copy(bufa[p], h_hbm.at[pl.ds(base, CHUNK)], wsa[p])
        return 0

    lax.fori_loop(0, TRIPS_P // 2, body, 0)
    for p in range(2):
        pltpu.make_async_copy(bufa[p], h_hbm.at[pl.ds(0, CHUNK)], wsa[p]).wait()


def _deg_body(packw_hbm, eww_hbm, zeros_hbm, dp_hbm,
              pkall, ewall, acc, ss0, ss1):
    c = lax.axis_index("c")
    s = lax.axis_index("s")
    w = s * NC + c
    ss = (ss0, ss1)

    pltpu.sync_copy(zeros_hbm.at[pl.ds(s * ROWS_PER_SUB, ROWS_PER_SUB)],
                    acc.at[pl.ds(s * ROWS_PER_SUB, ROWS_PER_SUB)])
    pltpu.sync_copy(packw_hbm.at[w], pkall)
    pltpu.sync_copy(eww_hbm.at[w], ewall)
    plsc.subcore_barrier()

    def body(k, _):
        for p in range(2):
            i = 2 * k + p

            @pl.when(i >= 2)
            def _():
                pltpu.make_async_copy(
                    ewall.at[i - 2], acc.at[pkall.at[i - 2, 0]], ss[p]).wait()

            pltpu.async_copy(ewall.at[i], acc.at[pkall.at[i, 0]], ss[p],
                             add=True)
        return 0

    lax.fori_loop(0, TRIPS_P // 2, body, 0)
    for p in range(2):
        pltpu.make_async_copy(ewall.at[0], acc.at[pkall.at[0, 0]], ss[p]).wait()
    plsc.subcore_barrier()
    pltpu.sync_copy(acc.at[pl.ds(s * ROWS_PER_SUB, ROWS_PER_SUB)],
                    dp_hbm.at[c, pl.ds(s * ROWS_PER_SUB, ROWS_PER_SUB)])


def _spmm_body(u_hbm, packw_hbm, eww_hbm, zeros_hbm, out_hbm,
               pk0, pk1, ewall, buf0, buf1, acc, gs0, gs1, ss0, ss1):
    c = lax.axis_index("c")
    s = lax.axis_index("s")
    w = s * NC + c
    pk = (pk0, pk1)
    buf = (buf0, buf1)
    gs = (gs0, gs1)
    ss = (ss0, ss1)

    pltpu.sync_copy(zeros_hbm.at[pl.ds(s * ROWS_PER_SUB, ROWS_PER_SUB)],
                    acc.at[pl.ds(s * ROWS_PER_SUB, ROWS_PER_SUB)])
    pltpu.sync_copy(eww_hbm.at[w], ewall)
    plsc.subcore_barrier()

    pltpu.sync_copy(packw_hbm.at[w, 0], pk0)
    pltpu.async_copy(u_hbm.at[pk0.at[0]], buf0, gs0)

    def body(k, _):
        for p in range(2):
            i = 2 * k + p
            q = 1 - p

            @pl.when(i + 1 < TRIPS_P)
            def _():
                pltpu.sync_copy(packw_hbm.at[w, i + 1], pk[q])

                @pl.when(i >= 1)
                def _():
                    pltpu.make_async_copy(
                        buf[q], acc.at[pk[q].at[1]], ss[q]).wait()

                pltpu.async_copy(u_hbm.at[pk[q].at[0]], buf[q], gs[q])

            pltpu.make_async_copy(u_hbm.at[pk[p].at[0]], buf[p], gs[p]).wait()

            def scale(g, _):
                wvec = ewall[i, pl.ds(g * 16, 16)]
                for l in range(16):
                    we = wvec[l]
                    e = g * 16 + l
                    for j in range(D // 16):
                        sl = pl.ds(j * 16, 16)
                        buf[p][e, sl] = buf[p][e, sl] * we
                return 0

            lax.fori_loop(0, CHUNK // 16, scale, 0)
            pltpu.async_copy(buf[p], acc.at[pk[p].at[1]], ss[p], add=True)
        return 0

    lax.fori_loop(0, TRIPS_P // 2, body, 0)
    for p in range(2):
        pltpu.make_async_copy(buf[p], acc.at[pk[p].at[1]], ss[p]).wait()
    plsc.subcore_barrier()
    pltpu.sync_copy(acc.at[pl.ds(s * ROWS_PER_SUB, ROWS_PER_SUB)],
                    out_hbm.at[c, pl.ds(s * ROWS_PER_SUB, ROWS_PER_SUB)])


@functools.lru_cache(maxsize=1)
def _sc_kernels():
    mesh = plsc.VectorSubcoreMesh(core_axis_name="c", subcore_axis_name="s",
                                  num_cores=NC, num_subcores=NS)
    gatherh = pl.kernel(
        _gatherh_body,
        out_type=jax.ShapeDtypeStruct((EPAD, D), jnp.float32),
        mesh=mesh,
        scratch_types=[
            pltpu.VMEM((TRIPS_P, 2, CHUNK), jnp.int32),
            pltpu.VMEM((CHUNK, D), jnp.float32),
            pltpu.VMEM((CHUNK, D), jnp.float32),
            pltpu.VMEM((CHUNK, D), jnp.float32),
            pltpu.VMEM((CHUNK, D), jnp.float32),
        ] + [pltpu.SemaphoreType.DMA] * 6,
    )
    deg = pl.kernel(
        _deg_body,
        out_type=jax.ShapeDtypeStruct((NC, NPAD), jnp.float32),
        mesh=mesh,
        scratch_types=[
            pltpu.VMEM((TRIPS_P, 2, CHUNK), jnp.int32),
            pltpu.VMEM((TRIPS_P, CHUNK), jnp.float32),
            pltpu.VMEM_SHARED((NPAD,), jnp.float32),
        ] + [pltpu.SemaphoreType.DMA] * 2,
    )
    spmm = pl.kernel(
        _spmm_body,
        out_type=jax.ShapeDtypeStruct((NC, NPAD, D), jnp.float32),
        mesh=mesh,
        scratch_types=[
            pltpu.VMEM((2, CHUNK), jnp.int32),
            pltpu.VMEM((2, CHUNK), jnp.int32),
            pltpu.VMEM((TRIPS_P, CHUNK), jnp.float32),
            pltpu.VMEM((CHUNK, D), jnp.float32),
            pltpu.VMEM((CHUNK, D), jnp.float32),
            pltpu.VMEM_SHARED((NPAD, D), jnp.float32),
        ] + [pltpu.SemaphoreType.DMA] * 4,
    )
    return gatherh, deg, spmm


_NBLK = 1000
_EBLK = 2000


def _rsqrt_precise(v):
    r = lax.rsqrt(v)
    return r * (1.5 - 0.5 * v * r * r)


def _tc_ab_body(x_ref, w1a_ref, w1b_ref, b1_ref, a_ref, b_ref):
    x = x_ref[...]
    a_ref[...] = jnp.dot(x, w1a_ref[...], preferred_element_type=jnp.float32,
                         precision=lax.Precision.HIGHEST) + b1_ref[...]
    b_ref[...] = jnp.dot(x, w1b_ref[...], preferred_element_type=jnp.float32,
                         precision=lax.Precision.HIGHEST)


def _tc_ab(x, w1a, w1b, b1):
    grid = (N // _NBLK,)
    return pl.pallas_call(
        _tc_ab_body,
        grid=grid,
        in_specs=[
            pl.BlockSpec((_NBLK, D), lambda i: (i, 0)),
            pl.BlockSpec((D, D), lambda i: (0, 0)),
            pl.BlockSpec((D, D), lambda i: (0, 0)),
            pl.BlockSpec((1, D), lambda i: (0, 0)),
        ],
        out_specs=[
            pl.BlockSpec((_NBLK, D), lambda i: (i, 0)),
            pl.BlockSpec((_NBLK, D), lambda i: (i, 0)),
        ],
        out_shape=[
            jax.ShapeDtypeStruct((N, D), jnp.float32),
            jax.ShapeDtypeStruct((N, D), jnp.float32),
        ],
    )(x, w1a, w1b, b1)


def _tc_edge_body(h_ref, g_ref, b_ref, w2_ref, b2_ref, s_ref):
    h = h_ref[...]
    mu = jnp.mean(h, axis=-1, keepdims=True)
    hc = h - mu
    var = jnp.mean(hc * hc, axis=-1, keepdims=True)
    hn = hc * _rsqrt_precise(var + 1e-5) * g_ref[...] + b_ref[...]
    hg = 0.5 * hn * (1.0 + lax.erf(hn * (1.0 / math.sqrt(2.0))))
    s = jnp.sum(hg * w2_ref[...], axis=-1, keepdims=True) + b2_ref[...]
    s_ref[...] = jax.nn.sigmoid(s)


def _tc_edge(h, ln_g, ln_b, w2row, b2):
    grid = (E // _EBLK,)
    return pl.pallas_call(
        _tc_edge_body,
        grid=grid,
        in_specs=[
            pl.BlockSpec((_EBLK, D), lambda i: (i, 0)),
            pl.BlockSpec((1, D), lambda i: (0, 0)),
            pl.BlockSpec((1, D), lambda i: (0, 0)),
            pl.BlockSpec((1, D), lambda i: (0, 0)),
            pl.BlockSpec((1, 1), lambda i: (0, 0)),
        ],
        out_specs=pl.BlockSpec((_EBLK, 1), lambda i: (i, 0)),
        out_shape=jax.ShapeDtypeStruct((E, 1), jnp.float32),
    )(h, ln_g, ln_b, w2row, b2)


def _tc_deg_body(dp0_ref, dp1_ref, x_ref, dis_ref, u_ref):
    deg = dp0_ref[...] + dp1_ref[...]
    pos = deg > 0
    dis = jnp.where(pos, _rsqrt_precise(jnp.where(pos, deg, 1.0)), 0.0)
    dis_ref[...] = dis
    u_ref[...] = x_ref[...] * dis


def _tc_deg(dp0, dp1, x):
    grid = (N // _NBLK,)
    return pl.pallas_call(
        _tc_deg_body,
        grid=grid,
        in_specs=[
            pl.BlockSpec((_NBLK, 1), lambda i: (i, 0)),
            pl.BlockSpec((_NBLK, 1), lambda i: (i, 0)),
            pl.BlockSpec((_NBLK, D), lambda i: (i, 0)),
        ],
        out_specs=[
            pl.BlockSpec((_NBLK, 1), lambda i: (i, 0)),
            pl.BlockSpec((_NBLK, D), lambda i: (i, 0)),
        ],
        out_shape=[
            jax.ShapeDtypeStruct((N, 1), jnp.float32),
            jax.ShapeDtypeStruct((N, D), jnp.float32),
        ],
    )(dp0, dp1, x)


def _tc_comb_body(alpha, beta, p_ref, dis_ref, prev_ref, t_ref, u_ref):
    dis = dis_ref[...]
    t = alpha * dis * (p_ref[0] + p_ref[1]) + beta * prev_ref[...]
    t_ref[...] = t
    u_ref[...] = dis * t


def _tc_comb(alpha, beta, p, dis, prev):
    grid = (N // _NBLK,)
    return pl.pallas_call(
        functools.partial(_tc_comb_body, alpha, beta),
        grid=grid,
        in_specs=[
            pl.BlockSpec((2, _NBLK, D), lambda i: (0, i, 0)),
            pl.BlockSpec((_NBLK, 1), lambda i: (i, 0)),
            pl.BlockSpec((_NBLK, D), lambda i: (i, 0)),
        ],
        out_specs=[
            pl.BlockSpec((_NBLK, D), lambda i: (i, 0)),
            pl.BlockSpec((_NBLK, D), lambda i: (i, 0)),
        ],
        out_shape=[
            jax.ShapeDtypeStruct((N, D), jnp.float32),
            jax.ShapeDtypeStruct((N, D), jnp.float32),
        ],
    )(p, dis, prev)


def _tc_final_body(x_ref, t1_ref, p2_ref, p3_ref,
                   dis_ref, w2_ref, b2_ref, w3_ref, b3_ref, w4_ref, b4_ref,
                   awt_ref, ab_ref, out_ref):
    dis = dis_ref[...]
    t0 = x_ref[...]
    t1 = t1_ref[...]
    t2 = -2.0 * dis * (p2_ref[0] + p2_ref[1]) - t0
    t3 = -2.0 * dis * (p3_ref[0] + p3_ref[1]) - t1

    def mm(a, w):
        return jnp.dot(a, w, preferred_element_type=jnp.float32,
                       precision=lax.Precision.HIGHEST)

    f2 = mm(t0, w2_ref[0]) + mm(t1, w2_ref[1]) + b2_ref[...]
    f3 = mm(t0, w3_ref[0]) + mm(t1, w3_ref[1]) + mm(t2, w3_ref[2]) + b3_ref[...]
    f4 = (mm(t0, w4_ref[0]) + mm(t1, w4_ref[1]) + mm(t2, w4_ref[2])
          + mm(t3, w4_ref[3]) + b4_ref[...])

    awt = awt_ref[...]
    fs = (f2, f3, f4)
    s = []
    for j in range(3):
        acc = ab_ref[0, j]
        tot = None
        for k in range(3):
            term = fs[k] * awt[j:j + 1, k * D:(k + 1) * D]
            tot = term if tot is None else tot + term
        s.append(jnp.sum(tot, axis=-1, keepdims=True) + acc)
    m = jnp.maximum(jnp.maximum(s[0], s[1]), s[2])
    e0 = jnp.exp(s[0] - m)
    e1 = jnp.exp(s[1] - m)
    e2 = jnp.exp(s[2] - m)
    out_ref[...] = (f2 * e0 + f3 * e1 + f4 * e2) / (e0 + e1 + e2)


def _tc_final(x, t1, p2, p3, dis, w2, b2, w3, b3, w4, b4, awt, ab):
    grid = (N // _NBLK,)
    full = lambda shape: pl.BlockSpec(shape, lambda i: tuple(0 for _ in shape))
    blk = pl.BlockSpec((_NBLK, D), lambda i: (i, 0))
    pblk = pl.BlockSpec((2, _NBLK, D), lambda i: (0, i, 0))
    return pl.pallas_call(
        _tc_final_body,
        grid=grid,
        in_specs=[
            blk, blk, pblk, pblk,
            pl.BlockSpec((_NBLK, 1), lambda i: (i, 0)),
            full((2, D, D)), full((1, D)),
            full((3, D, D)), full((1, D)),
            full((4, D, D)), full((1, D)),
            full((3, 3 * D)),
            pl.BlockSpec((1, 3), lambda i: (0, 0), memory_space=pltpu.SMEM),
        ],
        out_specs=blk,
        out_shape=jax.ShapeDtypeStruct((N, D), jnp.float32),
    )(x, t1, p2, p3, dis, w2, b2, w3, b3, w4, b4, awt, ab)


def kernel(x, edge_index, ss_w1, ss_b1, ss_ln_g, ss_ln_b, ss_w2, ss_b2,
           cheb2_w, cheb2_b, cheb3_w, cheb3_b, cheb4_w, cheb4_b, att_w, att_b):
    row = edge_index[0]
    col = edge_index[1]
    sc_gatherh, sc_deg, sc_spmm = _sc_kernels()

    fill = (jnp.arange(EPAD - E, dtype=jnp.int32) * 97) % N
    row_p = jnp.concatenate([row, fill])
    col_p = jnp.concatenate([col, fill])
    pack = jnp.stack([row_p.reshape(NCHUNKS_P, CHUNK),
                      col_p.reshape(NCHUNKS_P, CHUNK)], axis=1)
    packw = pack.reshape(TRIPS_P, NW, 2, CHUNK).transpose(1, 0, 2, 3)

    a_nodes, b_nodes = _tc_ab(x, ss_w1[:D], ss_w1[D:], ss_b1.reshape(1, D))
    h_edges = sc_gatherh(a_nodes, b_nodes, packw)
    edge_scores = _tc_edge(h_edges, ss_ln_g.reshape(1, D),
                           ss_ln_b.reshape(1, D), ss_w2.reshape(1, D),
                           ss_b2.reshape(1, 1))
    ew = edge_scores.reshape(E)
    ew_p = jnp.concatenate([ew, jnp.zeros((EPAD - E,), jnp.float32)])
    eww = ew_p.reshape(TRIPS_P, NW, CHUNK).transpose(1, 0, 2)

    zeros1 = jnp.zeros((NPAD,), jnp.float32)
    zeros2 = jnp.zeros((NPAD, D), jnp.float32)

    dp = sc_deg(packw, eww, zeros1)
    dis, u1 = _tc_deg(dp[0, :N].reshape(N, 1), dp[1, :N].reshape(N, 1), x)

    p1 = sc_spmm(u1, packw, eww, zeros2)
    t1, u2 = _tc_comb(-1.0, 0.0, p1, dis, x)
    p2 = sc_spmm(u2, packw, eww, zeros2)
    _, u3 = _tc_comb(-2.0, -1.0, p2, dis, x)
    p3 = sc_spmm(u3, packw, eww, zeros2)

    out = _tc_final(x, t1, p2, p3, dis,
                    cheb2_w, cheb2_b.reshape(1, D),
                    cheb3_w, cheb3_b.reshape(1, D),
                    cheb4_w, cheb4_b.reshape(1, D),
                    att_w.T, att_b.reshape(1, 3))
    return (out, edge_scores)
